# baseline (speedup 1.0000x reference)
import functools

import jax
import jax.numpy as jnp
from jax import lax
from jax.experimental import pallas as pl
from jax.experimental.pallas import tpu as pltpu
from jax.experimental.pallas import tpu_sc as plsc

NUM_EMBEDDINGS = 1000000
FEATURES = 64
BATCH = 4096
SEQ = 20
TOTAL = BATCH * SEQ
CLIP = 127.0
W = 512
NBLK = 977
H = NBLK * W
CHUNK = 128


def _tc_quant_body(x0_ref, x1_ref, o_ref):
    def dq(x):
        m = jnp.maximum(jnp.max(jnp.abs(x), axis=0, keepdims=True), 1e-9)
        scale = CLIP / m
        q = jnp.round(jnp.clip(x * scale, -CLIP, CLIP))
        return (q * (m * (1.0 / CLIP))).T

    o_ref[:, 0:FEATURES] = dq(x0_ref[...])
    o_ref[:, FEATURES:2 * FEATURES] = dq(x1_ref[...])


def _dequant_table(emb_t):
    return pl.pallas_call(
        _tc_quant_body,
        grid=(NBLK,),
        in_specs=[
            pl.BlockSpec((FEATURES, W), lambda g: (0, g)),
            pl.BlockSpec((FEATURES, W), lambda g: (0, NBLK + g)),
        ],
        out_specs=pl.BlockSpec((W, 2 * FEATURES), lambda g: (g, 0)),
        out_shape=jax.ShapeDtypeStruct((H, 2 * FEATURES), jnp.float32),
    )(emb_t, emb_t)


def _sc_body(nc, chunks, table_hbm, idx_hbm, out_hbm, idx_v, rows_v, sem):
    wid = lax.axis_index("s") * nc + lax.axis_index("c")
    pltpu.sync_copy(idx_hbm.at[wid], idx_v)

    def chunk_step(j, carry):
        pltpu.async_copy(table_hbm.at[idx_v.at[j]], rows_v, sem).wait()
        pltpu.sync_copy(
            rows_v, out_hbm.at[pl.ds((wid * chunks + j) * CHUNK, CHUNK)])
        return carry

    lax.fori_loop(0, chunks, chunk_step, 0)


def kernel(inputs, embedding):
    info = plsc.get_sparse_core_info()
    nc, ns = info.num_cores, info.num_subcores
    nw = nc * ns
    chunks = TOTAL // (nw * CHUNK)

    table = _dequant_table(embedding.T).reshape(2 * H, FEATURES)
    hi = (inputs >= H).astype(jnp.int32)
    rows = (2 * (inputs - hi * H) + hi).reshape(nw, chunks, CHUNK)

    mesh = plsc.VectorSubcoreMesh(core_axis_name="c", subcore_axis_name="s")
    k = pl.kernel(
        functools.partial(_sc_body, nc, chunks),
        mesh=mesh,
        out_type=jax.ShapeDtypeStruct((TOTAL, FEATURES), jnp.float32),
        scratch_types=[
            pltpu.VMEM((chunks, CHUNK), jnp.int32),
            pltpu.VMEM((CHUNK, FEATURES), jnp.float32),
            pltpu.SemaphoreType.DMA,
        ],
        compiler_params=pltpu.CompilerParams(
            use_tc_tiling_on_sc=False, needs_layout_passes=False),
    )
    out = k(table, rows)
    return out.reshape(BATCH, SEQ, FEATURES)


# speedup vs baseline: 1.8479x; 1.8479x over previous
import functools

import jax
import jax.numpy as jnp
from jax import lax
from jax.experimental import pallas as pl
from jax.experimental.pallas import tpu as pltpu
from jax.experimental.pallas import tpu_sc as plsc

NUM_EMBEDDINGS = 1000000
FEATURES = 64
BATCH = 4096
SEQ = 20
TOTAL = BATCH * SEQ
CLIP = 127.0
W = 2048
NBLK = 245
H = NBLK * W
CLAMPB = (NUM_EMBEDDINGS - W) // W
TAILSTART = (CLAMPB + 1) * W
TAIL = NUM_EMBEDDINGS - TAILSTART
CHUNK = 128


def _dq(x):
    m = jnp.maximum(jnp.max(jnp.abs(x), axis=0, keepdims=True), 1e-9)
    scale = CLIP / m
    q = jnp.round(jnp.clip(x * scale, -CLIP, CLIP))
    return (q * (m * (1.0 / CLIP))).T


def _tc_quant_body(x0_ref, x1_ref, tail_ref, o_ref):
    o_ref[:, 0:FEATURES] = _dq(x0_ref[...])
    o_ref[:, FEATURES:2 * FEATURES] = _dq(x1_ref[...])

    @pl.when(pl.program_id(0) == NBLK - 2)
    def _():
        o_ref[0:TAIL, FEATURES:2 * FEATURES] = _dq(tail_ref[...])


def _dequant_table(emb_t, tail_t):
    return pl.pallas_call(
        _tc_quant_body,
        grid=(NBLK,),
        in_specs=[
            pl.BlockSpec((FEATURES, W), lambda g: (0, g)),
            pl.BlockSpec((FEATURES, W),
                         lambda g: (0, jnp.minimum(NBLK + g, CLAMPB))),
            pl.BlockSpec((FEATURES, TAIL), lambda g: (0, 0)),
        ],
        out_specs=pl.BlockSpec((W, 2 * FEATURES), lambda g: (g, 0)),
        out_shape=jax.ShapeDtypeStruct((H, 2 * FEATURES), jnp.float32),
    )(emb_t, emb_t, tail_t)


def _sc_body(nc, chunks, table_hbm, idx_hbm, out_hbm, idx_v, rows_v, sem):
    wid = lax.axis_index("s") * nc + lax.axis_index("c")
    pltpu.sync_copy(idx_hbm.at[wid], idx_v)

    def chunk_step(j, carry):
        pltpu.async_copy(table_hbm.at[idx_v.at[j]], rows_v, sem).wait()
        pltpu.sync_copy(
            rows_v, out_hbm.at[pl.ds((wid * chunks + j) * CHUNK, CHUNK)])
        return carry

    lax.fori_loop(0, chunks, chunk_step, 0)


def kernel(inputs, embedding):
    info = plsc.get_sparse_core_info()
    nc, ns = info.num_cores, info.num_subcores
    nw = nc * ns
    chunks = TOTAL // (nw * CHUNK)

    table = _dequant_table(
        embedding.T, embedding[TAILSTART:].T).reshape(2 * H, FEATURES)
    hi = (inputs >= H).astype(jnp.int32)
    rows = (2 * (inputs - hi * H) + hi).reshape(nw, chunks, CHUNK)

    mesh = plsc.VectorSubcoreMesh(core_axis_name="c", subcore_axis_name="s")
    k = pl.kernel(
        functools.partial(_sc_body, nc, chunks),
        mesh=mesh,
        out_type=jax.ShapeDtypeStruct((TOTAL, FEATURES), jnp.float32),
        scratch_types=[
            pltpu.VMEM((chunks, CHUNK), jnp.int32),
            pltpu.VMEM((CHUNK, FEATURES), jnp.float32),
            pltpu.SemaphoreType.DMA,
        ],
        compiler_params=pltpu.CompilerParams(
            use_tc_tiling_on_sc=False, needs_layout_passes=False),
    )
    out = k(table, rows)
    return out.reshape(BATCH, SEQ, FEATURES)
